# baseline (speedup 1.0000x reference)
import functools

import jax
import jax.numpy as jnp
from jax import lax
from jax.experimental import pallas as pl
from jax.experimental.pallas import tpu as pltpu
from jax.experimental.pallas import tpu_sc as plsc

NC, NS, LANES = 2, 16, 16
NW = NC * NS

EMB = 64
ROW = 128
SEQ = 200
HIDDEN = 256
NUM_LABELS = 10


def _dup_body(x_ref, o_ref):
    x = x_ref[...]
    o_ref[...] = jnp.concatenate([x, x], axis=1)


def _tc_dup(emb):
    v_total = emb.shape[0]
    bt = 2048
    grid = (pl.cdiv(v_total, bt),)
    return pl.pallas_call(
        _dup_body,
        grid=grid,
        in_specs=[pl.BlockSpec((bt, EMB), lambda i: (i, 0))],
        out_specs=pl.BlockSpec((bt, ROW), lambda i: (i, 0)),
        out_shape=jax.ShapeDtypeStruct((v_total, ROW), jnp.float32),
    )(emb)


def _gather_copies(eh, idxs_v, e, rows_buf, sem):
    return (
        pltpu.make_async_copy(eh.at[idxs_v.at[pl.ds(e * SEQ, 128)]],
                              rows_buf.at[pl.ds(0, 128)], sem),
        pltpu.make_async_copy(eh.at[idxs_v.at[pl.ds(e * SEQ + 128, SEQ - 128)]],
                              rows_buf.at[pl.ds(128, SEQ - 128)], sem),
    )


def _reduce_rows(rows_buf, out_v, e):
    nacc = EMB // LANES
    unroll = 8

    def red(t, accs):
        for u in range(unroll):
            s = t * unroll + u
            accs = tuple(a + rows_buf[s, pl.ds(LANES * j, LANES)]
                         for j, a in enumerate(accs))
        return accs

    accs = lax.fori_loop(
        0, SEQ // unroll, red,
        tuple(jnp.zeros((LANES,), jnp.float32) for _ in range(nacc)))
    for j in range(nacc):
        out_v[pl.ds(e * EMB + LANES * j, LANES)] = accs[j]


def _pool_body(xh, eh, oh, idxs_v, rows_a, rows_b, out_v,
               sem_i, sem_a, sem_b):
    batch = oh.shape[0] // EMB
    bpw = batch // NW
    wid = lax.axis_index("s") * NC + lax.axis_index("c")
    base = wid * bpw
    bufs = (rows_a, rows_b)
    sems = (sem_a, sem_b)
    pltpu.async_copy(xh.at[pl.ds(base * SEQ, bpw * SEQ)], idxs_v, sem_i).wait()
    for c in _gather_copies(eh, idxs_v, 0, bufs[0], sems[0]):
        c.start()

    @pl.loop(0, bpw // 2)
    def _(i):
        for p in range(2):
            e = 2 * i + p
            for c in _gather_copies(eh, idxs_v, e, bufs[p], sems[p]):
                c.wait()
            nxt = e + 1

            @pl.when(nxt < bpw)
            def _():
                for c in _gather_copies(eh, idxs_v, nxt,
                                        bufs[1 - p], sems[1 - p]):
                    c.start()

            _reduce_rows(bufs[p], out_v, e)

    pltpu.sync_copy(out_v, oh.at[pl.ds(base * EMB, bpw * EMB)])


def _sc_pool_one(xf, emb128, batch):
    bpw = batch // NW
    mesh = plsc.VectorSubcoreMesh(core_axis_name="c", subcore_axis_name="s")
    return pl.kernel(
        _pool_body,
        out_type=jax.ShapeDtypeStruct((batch * EMB,), jnp.float32),
        mesh=mesh,
        compiler_params=pltpu.CompilerParams(use_tc_tiling_on_sc=True),
        scratch_types=[
            pltpu.VMEM((bpw * SEQ,), jnp.int32),
            pltpu.VMEM((SEQ, ROW), jnp.float32),
            pltpu.VMEM((SEQ, ROW), jnp.float32),
            pltpu.VMEM((bpw * EMB,), jnp.float32),
            pltpu.SemaphoreType.DMA,
            pltpu.SemaphoreType.DMA,
            pltpu.SemaphoreType.DMA,
        ],
    )(xf, emb128)


def _mlp_body(p0_ref, p1_ref, p2_ref, w1_ref, b1_ref, w2_ref, b2_ref, o_ref):
    x = jnp.concatenate([p0_ref[...], p1_ref[...], p2_ref[...]], axis=1)
    h = jnp.dot(x, w1_ref[...], preferred_element_type=jnp.float32)
    h = h * (1.0 / SEQ) + b1_ref[...]
    h = jnp.maximum(h, 0.0)
    o_ref[...] = (jnp.dot(h, w2_ref[...], preferred_element_type=jnp.float32)
                  + b2_ref[...])


def _tc_mlp(p0, p1, p2, W1, b1, W2, b2):
    batch = p0.shape[0]
    bt = 512
    grid = (batch // bt,)
    return pl.pallas_call(
        _mlp_body,
        grid=grid,
        in_specs=[
            pl.BlockSpec((bt, EMB), lambda i: (i, 0)),
            pl.BlockSpec((bt, EMB), lambda i: (i, 0)),
            pl.BlockSpec((bt, EMB), lambda i: (i, 0)),
            pl.BlockSpec((3 * EMB, HIDDEN), lambda i: (0, 0)),
            pl.BlockSpec((1, HIDDEN), lambda i: (0, 0)),
            pl.BlockSpec((HIDDEN, NUM_LABELS), lambda i: (0, 0)),
            pl.BlockSpec((1, NUM_LABELS), lambda i: (0, 0)),
        ],
        out_specs=pl.BlockSpec((bt, NUM_LABELS), lambda i: (i, 0)),
        out_shape=jax.ShapeDtypeStruct((batch, NUM_LABELS), jnp.float32),
    )(p0, p1, p2, W1, b1, W2, b2)


def kernel(x0, x1, x2, x3, emb_word, emb_bi, emb_tri, W1, b1, W2, b2):
    del x1
    batch = x0.shape[0]
    p0 = _sc_pool_one(x0.astype(jnp.int32).reshape(-1),
                      _tc_dup(emb_word), batch).reshape(batch, EMB)
    p1 = _sc_pool_one(x2.reshape(-1), _tc_dup(emb_bi), batch
                      ).reshape(batch, EMB)
    p2 = _sc_pool_one(x3.reshape(-1), _tc_dup(emb_tri), batch
                      ).reshape(batch, EMB)
    return _tc_mlp(p0, p1, p2, W1, b1.reshape(1, HIDDEN),
                   W2, b2.reshape(1, NUM_LABELS))


# speedup vs baseline: 1.8174x; 1.8174x over previous
import functools

import jax
import jax.numpy as jnp
from jax import lax
from jax.experimental import pallas as pl
from jax.experimental.pallas import tpu as pltpu
from jax.experimental.pallas import tpu_sc as plsc

NC, NS, LANES = 2, 16, 16
NW = NC * NS

EMB = 64
SEQ = 200
HIDDEN = 256
NUM_LABELS = 10


def _gather_copies(eh, idxs_v, e, rows_buf, sem):
    return (
        pltpu.make_async_copy(eh.at[idxs_v.at[e, pl.ds(0, 128)]],
                              rows_buf.at[pl.ds(0, 128)], sem),
        pltpu.make_async_copy(eh.at[idxs_v.at[e, pl.ds(128, SEQ - 128)]],
                              rows_buf.at[pl.ds(128, SEQ - 128)], sem),
    )


def _reduce_rows(rows_buf, out_v, e):
    nacc = EMB // LANES
    unroll = 8

    def red(t, accs):
        for u in range(unroll):
            s = t * unroll + u
            accs = tuple(a + rows_buf[s, pl.ds(LANES * j, LANES)]
                         for j, a in enumerate(accs))
        return accs

    accs = lax.fori_loop(
        0, SEQ // unroll, red,
        tuple(jnp.zeros((LANES,), jnp.float32) for _ in range(nacc)))
    for j in range(nacc):
        out_v[e, pl.ds(LANES * j, LANES)] = accs[j]


NBUF = 4


def _pool_body(xh, eh, oh, idxs_v, rows_0, rows_1, rows_2, rows_3, out_v,
               sem_i, sem_0, sem_1, sem_2, sem_3):
    batch = xh.shape[0]
    bpw = batch // NW
    wid = lax.axis_index("s") * NC + lax.axis_index("c")
    base = wid * bpw
    bufs = (rows_0, rows_1, rows_2, rows_3)
    sems = (sem_0, sem_1, sem_2, sem_3)
    pltpu.async_copy(xh.at[pl.ds(base, bpw)], idxs_v, sem_i).wait()
    for e0 in range(NBUF - 1):
        for c in _gather_copies(eh, idxs_v, e0, bufs[e0], sems[e0]):
            c.start()

    @pl.loop(0, bpw // NBUF)
    def _(i):
        for p in range(NBUF):
            e = NBUF * i + p
            nxt = e + NBUF - 1

            @pl.when(nxt < bpw)
            def _():
                q = (p + NBUF - 1) % NBUF
                for c in _gather_copies(eh, idxs_v, nxt, bufs[q], sems[q]):
                    c.start()

            for c in _gather_copies(eh, idxs_v, e, bufs[p], sems[p]):
                c.wait()
            _reduce_rows(bufs[p], out_v, e)

    pltpu.sync_copy(out_v, oh.at[pl.ds(base, bpw)])


def _sc_pool_one(x, emb):
    batch = x.shape[0]
    bpw = batch // NW
    mesh = plsc.VectorSubcoreMesh(core_axis_name="c", subcore_axis_name="s")
    return pl.kernel(
        _pool_body,
        out_type=jax.ShapeDtypeStruct((batch, EMB), jnp.float32),
        mesh=mesh,
        compiler_params=pltpu.CompilerParams(use_tc_tiling_on_sc=False),
        scratch_types=(
            [pltpu.VMEM((bpw, SEQ), jnp.int32)]
            + [pltpu.VMEM((SEQ, EMB), jnp.float32) for _ in range(NBUF)]
            + [pltpu.VMEM((bpw, EMB), jnp.float32)]
            + [pltpu.SemaphoreType.DMA for _ in range(NBUF + 1)]
        ),
    )(x, emb)


def _mlp_body(p0_ref, p1_ref, p2_ref, w1_ref, b1_ref, w2_ref, b2_ref, o_ref):
    x = jnp.concatenate([p0_ref[...], p1_ref[...], p2_ref[...]], axis=1)
    h = jnp.dot(x, w1_ref[...], preferred_element_type=jnp.float32)
    h = h * (1.0 / SEQ) + b1_ref[...]
    h = jnp.maximum(h, 0.0)
    o_ref[...] = (jnp.dot(h, w2_ref[...], preferred_element_type=jnp.float32)
                  + b2_ref[...])


def _tc_mlp(p0, p1, p2, W1, b1, W2, b2):
    batch = p0.shape[0]
    bt = 512
    grid = (batch // bt,)
    return pl.pallas_call(
        _mlp_body,
        grid=grid,
        in_specs=[
            pl.BlockSpec((bt, EMB), lambda i: (i, 0)),
            pl.BlockSpec((bt, EMB), lambda i: (i, 0)),
            pl.BlockSpec((bt, EMB), lambda i: (i, 0)),
            pl.BlockSpec((3 * EMB, HIDDEN), lambda i: (0, 0)),
            pl.BlockSpec((1, HIDDEN), lambda i: (0, 0)),
            pl.BlockSpec((HIDDEN, NUM_LABELS), lambda i: (0, 0)),
            pl.BlockSpec((1, NUM_LABELS), lambda i: (0, 0)),
        ],
        out_specs=pl.BlockSpec((bt, NUM_LABELS), lambda i: (i, 0)),
        out_shape=jax.ShapeDtypeStruct((batch, NUM_LABELS), jnp.float32),
    )(p0, p1, p2, W1, b1, W2, b2)


def kernel(x0, x1, x2, x3, emb_word, emb_bi, emb_tri, W1, b1, W2, b2):
    del x1
    p0 = _sc_pool_one(x0.astype(jnp.int32), emb_word)
    p1 = _sc_pool_one(x2, emb_bi)
    p2 = _sc_pool_one(x3, emb_tri)
    return _tc_mlp(p0, p1, p2, W1, b1.reshape(1, HIDDEN),
                   W2, b2.reshape(1, NUM_LABELS))
